# baseline (speedup 1.0000x reference)
import functools

import jax
import jax.numpy as jnp
from jax import lax
from jax.experimental import pallas as pl
from jax.experimental.pallas import tpu as pltpu
from jax.experimental.pallas import tpu_sc as plsc

_NUM_CODES = 8192
_CODE_DIM = 256
_N_TOKENS = 16384
_HALF = _N_TOKENS // 2
_BM = 512
_SCALE = 1.25 / (_N_TOKENS * _CODE_DIM)
_GW = 128
_N_GROUPS = _NUM_CODES // _GW
_RS = 64
_CSQ_BOUND = 2.0 ** -18


def _dist_body(z_ref, cb_ref, csq_ref, idx_ref, loss_ref, acc_ref):
    i = pl.program_id(0)
    z = z_ref[...]
    cb = cb_ref[...]
    csq = csq_ref[...]
    zsq = jnp.sum(z * z, axis=1, keepdims=True)
    mm2 = lax.dot_general(z + z, cb, (((1,), (1,)), ((), ())),
                          preferred_element_type=jnp.float32)

    lane = lax.broadcasted_iota(jnp.int32, (_RS, _GW), 1)

    def argmin_pass(fast):
        part = None
        for r in range(0, _BM, _RS):
            zsq_r = zsq[r:r + _RS]

            def dist_g(g):
                m = mm2[r:r + _RS, g * _GW:(g + 1) * _GW]
                if fast:
                    return zsq_r - m
                return (zsq_r + csq[:, g * _GW:(g + 1) * _GW]) - m

            rmin = dist_g(0)
            rgrp = jnp.zeros((_RS, _GW), jnp.int32)
            for g in range(1, _N_GROUPS):
                dg = dist_g(g)
                lt = dg < rmin
                rmin = jnp.where(lt, dg, rmin)
                rgrp = jnp.where(lt, g, rgrp)

            cidx = rgrp * _GW + lane
            dmin = jnp.min(rmin, axis=1, keepdims=True)
            cand = jnp.where(rmin == dmin, cidx, _NUM_CODES)
            idx_ref[r:r + _RS, :] = jnp.min(cand, axis=1, keepdims=True)
            ps = jnp.sum(dmin)
            part = ps if part is None else part + ps
        return part

    ok = jnp.logical_and(jnp.min(zsq) >= 129.0, jnp.max(csq) < _CSQ_BOUND)
    part = lax.cond(ok, lambda: argmin_pass(True), lambda: argmin_pass(False))

    @pl.when(i == 0)
    def _():
        acc_ref[0] = 0.0

    acc_ref[0] += part
    loss_ref[0] = acc_ref[0] * _SCALE


def _dist_call(z, codebook, csq_row, half):
    steps = _HALF // _BM
    off = half * steps
    return pl.pallas_call(
        _dist_body,
        grid=(steps,),
        in_specs=[
            pl.BlockSpec((_BM, _CODE_DIM), lambda i: (i + off, 0)),
            pl.BlockSpec((_NUM_CODES, _CODE_DIM), lambda i: (0, 0)),
            pl.BlockSpec((1, _NUM_CODES), lambda i: (0, 0)),
        ],
        out_specs=[
            pl.BlockSpec((_BM, 1), lambda i: (i, 0)),
            pl.BlockSpec(memory_space=pltpu.SMEM),
        ],
        out_shape=[
            jax.ShapeDtypeStruct((_HALF, 1), jnp.int32),
            jax.ShapeDtypeStruct((1,), jnp.float32),
        ],
        scratch_shapes=[pltpu.SMEM((1,), jnp.float32)],
    )(z, codebook, csq_row)


_N_WORKERS = 32
_B_PER_W = _HALF // _N_WORKERS
_CHUNK = 128


def _gather_body(idx_hbm, cb_hbm, out_hbm, idx_v, rows_v, sem):
    wid = lax.axis_index("s") * 2 + lax.axis_index("c")
    for c in range(_B_PER_W // _CHUNK):
        base = wid * _B_PER_W + c * _CHUNK
        pltpu.sync_copy(idx_hbm.at[pl.ds(base, _CHUNK)], idx_v)
        pltpu.async_copy(cb_hbm.at[idx_v], rows_v, sem).wait()
        pltpu.sync_copy(rows_v, out_hbm.at[pl.ds(base, _CHUNK)])


def _gather_rows(indices, codebook):
    mesh = plsc.VectorSubcoreMesh(core_axis_name="c", subcore_axis_name="s")
    gk = functools.partial(
        pl.kernel,
        mesh=mesh,
        out_type=jax.ShapeDtypeStruct((_HALF, _CODE_DIM), jnp.float32),
        scratch_types=[
            pltpu.VMEM((_CHUNK,), jnp.int32),
            pltpu.VMEM((_CHUNK, _CODE_DIM), jnp.float32),
            pltpu.SemaphoreType.DMA,
        ],
    )(_gather_body)
    return gk(indices, codebook)


def kernel(z, codebook):
    csq_row = jnp.sum(codebook * codebook, axis=1, keepdims=True).reshape(
        1, _NUM_CODES)
    idx_a, loss_a = _dist_call(z, codebook, csq_row, 0)
    zq_a = _gather_rows(idx_a.reshape(_HALF), codebook)
    idx_b, loss_b = _dist_call(z, codebook, csq_row, 1)
    zq_b = _gather_rows(idx_b.reshape(_HALF), codebook)
    indices = jnp.concatenate([idx_a.reshape(_HALF), idx_b.reshape(_HALF)])
    z_q = jnp.concatenate([zq_a, zq_b], axis=0)
    loss = loss_a[0] + loss_b[0]
    return (z_q, indices, loss)


# speedup vs baseline: 1.2515x; 1.2515x over previous
import functools

import jax
import jax.numpy as jnp
from jax import lax
from jax.experimental import pallas as pl
from jax.experimental.pallas import tpu as pltpu
from jax.experimental.pallas import tpu_sc as plsc

_NUM_CODES = 8192
_CODE_DIM = 256
_N_TOKENS = 16384
_HALF = _N_TOKENS // 2
_BM = 512
_SCALE = 1.25 / (_N_TOKENS * _CODE_DIM)
_GW = 128
_N_GROUPS = _NUM_CODES // _GW
_RS = 64
_CSQ_BOUND = 2.0 ** -18


def _make_dist_body(fast):
    def body(z_ref, cb_ref, csq_ref, zsq_ref, idx_ref, loss_ref, acc_ref):
        i = pl.program_id(0)
        z = z_ref[...]
        cb = cb_ref[...]
        csq = csq_ref[...]
        mm2 = lax.dot_general(z + z, cb, (((1,), (1,)), ((), ())),
                              preferred_element_type=jnp.float32)

        lane = lax.broadcasted_iota(jnp.int32, (_RS, _GW), 1)
        part = None
        for r in range(0, _BM, _RS):
            zsq_r = zsq_ref[r:r + _RS]

            def dist_g(g):
                m = mm2[r:r + _RS, g * _GW:(g + 1) * _GW]
                if fast:
                    return zsq_r - m
                return (zsq_r + csq[:, g * _GW:(g + 1) * _GW]) - m

            rmin = dist_g(0)
            rgrp = jnp.zeros((_RS, _GW), jnp.int32)
            for g in range(1, _N_GROUPS):
                dg = dist_g(g)
                lt = dg < rmin
                rmin = jnp.where(lt, dg, rmin)
                rgrp = jnp.where(lt, g, rgrp)

            cidx = rgrp * _GW + lane
            dmin = jnp.min(rmin, axis=1, keepdims=True)
            cand = jnp.where(rmin == dmin, cidx, _NUM_CODES)
            idx_ref[r:r + _RS, :] = jnp.min(cand, axis=1, keepdims=True)
            ps = jnp.sum(dmin)
            part = ps if part is None else part + ps

        @pl.when(i == 0)
        def _():
            acc_ref[0] = 0.0

        acc_ref[0] += part
        loss_ref[0] = acc_ref[0] * _SCALE

    return body


def _dist_call(z, codebook, csq_row, zsq, half, fast):
    steps = _HALF // _BM
    off = half * steps
    return pl.pallas_call(
        _make_dist_body(fast),
        grid=(steps,),
        in_specs=[
            pl.BlockSpec((_BM, _CODE_DIM), lambda i: (i + off, 0)),
            pl.BlockSpec((_NUM_CODES, _CODE_DIM), lambda i: (0, 0)),
            pl.BlockSpec((1, _NUM_CODES), lambda i: (0, 0)),
            pl.BlockSpec((_BM, 1), lambda i: (i + off, 0)),
        ],
        out_specs=[
            pl.BlockSpec((_BM, 1), lambda i: (i, 0)),
            pl.BlockSpec(memory_space=pltpu.SMEM),
        ],
        out_shape=[
            jax.ShapeDtypeStruct((_HALF, 1), jnp.int32),
            jax.ShapeDtypeStruct((1,), jnp.float32),
        ],
        scratch_shapes=[pltpu.SMEM((1,), jnp.float32)],
    )(z, codebook, csq_row, zsq)


_N_WORKERS = 32
_B_PER_W = _HALF // _N_WORKERS
_CHUNK = 128


def _gather_body(idx_hbm, cb_hbm, out_hbm, idx_v, rows_v, sem):
    wid = lax.axis_index("s") * 2 + lax.axis_index("c")
    for c in range(_B_PER_W // _CHUNK):
        base = wid * _B_PER_W + c * _CHUNK
        pltpu.sync_copy(idx_hbm.at[pl.ds(base, _CHUNK)], idx_v)
        pltpu.async_copy(cb_hbm.at[idx_v], rows_v, sem).wait()
        pltpu.sync_copy(rows_v, out_hbm.at[pl.ds(base, _CHUNK)])


def _gather_rows(indices, codebook):
    mesh = plsc.VectorSubcoreMesh(core_axis_name="c", subcore_axis_name="s")
    gk = functools.partial(
        pl.kernel,
        mesh=mesh,
        out_type=jax.ShapeDtypeStruct((_HALF, _CODE_DIM), jnp.float32),
        scratch_types=[
            pltpu.VMEM((_CHUNK,), jnp.int32),
            pltpu.VMEM((_CHUNK, _CODE_DIM), jnp.float32),
            pltpu.SemaphoreType.DMA,
        ],
    )(_gather_body)
    return gk(indices, codebook)


def _run(z, codebook, csq_row, zsq, fast):
    idx_a, loss_a = _dist_call(z, codebook, csq_row, zsq, 0, fast)
    zq_a = _gather_rows(idx_a.reshape(_HALF), codebook)
    idx_b, loss_b = _dist_call(z, codebook, csq_row, zsq, 1, fast)
    zq_b = _gather_rows(idx_b.reshape(_HALF), codebook)
    indices = jnp.concatenate([idx_a.reshape(_HALF), idx_b.reshape(_HALF)])
    z_q = jnp.concatenate([zq_a, zq_b], axis=0)
    loss = loss_a[0] + loss_b[0]
    return (z_q, indices, loss)


def kernel(z, codebook):
    csq_col = jnp.sum(codebook * codebook, axis=1, keepdims=True)
    csq_row = csq_col.reshape(1, _NUM_CODES)
    zsq = jnp.sum(z * z, axis=1, keepdims=True)
    fast_ok = jnp.logical_and(jnp.min(zsq) >= 129.0,
                              jnp.max(csq_col) < _CSQ_BOUND)
    return lax.cond(
        fast_ok,
        lambda: _run(z, codebook, csq_row, zsq, True),
        lambda: _run(z, codebook, csq_row, zsq, False),
    )


# speedup vs baseline: 1.3733x; 1.0973x over previous
import functools

import jax
import jax.numpy as jnp
from jax import lax
from jax.experimental import pallas as pl
from jax.experimental.pallas import tpu as pltpu
from jax.experimental.pallas import tpu_sc as plsc

_NUM_CODES = 8192
_CODE_DIM = 256
_N_TOKENS = 16384
_HALF = _N_TOKENS // 2
_BM = 512
_SCALE = 1.25 / (_N_TOKENS * _CODE_DIM)
_GW = 128
_N_GROUPS = _NUM_CODES // _GW
_RS = 64
_CSQ_BOUND = 2.0 ** -18


def _make_dist_body(fast):
    def body(z_ref, cb_ref, csq_ref, zsq_ref, idx_ref, loss_ref, acc_ref):
        i = pl.program_id(0)
        z = z_ref[...]
        cb = cb_ref[...]
        csq = csq_ref[...]
        mm2 = lax.dot_general(z + z, cb, (((1,), (1,)), ((), ())),
                              preferred_element_type=jnp.float32)

        lane = lax.broadcasted_iota(jnp.int32, (_RS, _GW), 1)
        part = None
        for r in range(0, _BM, _RS):
            zsq_r = zsq_ref[r:r + _RS]

            def dist_g(g):
                m = mm2[r:r + _RS, g * _GW:(g + 1) * _GW]
                if fast:
                    return zsq_r - m
                return (zsq_r + csq[:, g * _GW:(g + 1) * _GW]) - m

            rmin = dist_g(0)
            rgrp = jnp.zeros((_RS, _GW), jnp.int32)
            for g in range(1, _N_GROUPS):
                dg = dist_g(g)
                lt = dg < rmin
                rmin = jnp.where(lt, dg, rmin)
                rgrp = jnp.where(lt, g, rgrp)

            cidx = rgrp * _GW + lane
            dmin = jnp.min(rmin, axis=1, keepdims=True)
            cand = jnp.where(rmin == dmin, cidx, _NUM_CODES)
            idx_ref[r:r + _RS, :] = jnp.min(cand, axis=1, keepdims=True)
            ps = jnp.sum(dmin)
            part = ps if part is None else part + ps

        @pl.when(i == 0)
        def _():
            acc_ref[0] = 0.0

        acc_ref[0] += part
        loss_ref[0] = acc_ref[0] * _SCALE

    return body


def _dist_call(z, codebook, csq_row, zsq, fast):
    steps = _N_TOKENS // _BM
    return pl.pallas_call(
        _make_dist_body(fast),
        grid=(steps,),
        in_specs=[
            pl.BlockSpec((_BM, _CODE_DIM), lambda i: (i, 0)),
            pl.BlockSpec((_NUM_CODES, _CODE_DIM), lambda i: (0, 0)),
            pl.BlockSpec((1, _NUM_CODES), lambda i: (0, 0)),
            pl.BlockSpec((_BM, 1), lambda i: (i, 0)),
        ],
        out_specs=[
            pl.BlockSpec((_BM, 1), lambda i: (i, 0)),
            pl.BlockSpec(memory_space=pltpu.SMEM),
        ],
        out_shape=[
            jax.ShapeDtypeStruct((_N_TOKENS, 1), jnp.int32),
            jax.ShapeDtypeStruct((1,), jnp.float32),
        ],
        scratch_shapes=[pltpu.SMEM((1,), jnp.float32)],
    )(z, codebook, csq_row, zsq)


_N_WORKERS = 32
_B_PER_W = _N_TOKENS // _N_WORKERS
_CHUNK = 128


def _gather_body(idx_hbm, cb_hbm, out_hbm, idx_v, rows_v, sem):
    wid = lax.axis_index("s") * 2 + lax.axis_index("c")
    for c in range(_B_PER_W // _CHUNK):
        base = wid * _B_PER_W + c * _CHUNK
        pltpu.sync_copy(idx_hbm.at[pl.ds(base, _CHUNK)], idx_v)
        pltpu.async_copy(cb_hbm.at[idx_v], rows_v, sem).wait()
        pltpu.sync_copy(rows_v, out_hbm.at[pl.ds(base, _CHUNK)])


def _gather_rows(indices, codebook):
    mesh = plsc.VectorSubcoreMesh(core_axis_name="c", subcore_axis_name="s")
    gk = functools.partial(
        pl.kernel,
        mesh=mesh,
        out_type=jax.ShapeDtypeStruct((_N_TOKENS, _CODE_DIM), jnp.float32),
        scratch_types=[
            pltpu.VMEM((_CHUNK,), jnp.int32),
            pltpu.VMEM((_CHUNK, _CODE_DIM), jnp.float32),
            pltpu.SemaphoreType.DMA,
        ],
    )(_gather_body)
    return gk(indices, codebook)


def _run(z, codebook, csq_row, zsq, fast):
    idx, loss = _dist_call(z, codebook, csq_row, zsq, fast)
    indices = idx.reshape(_N_TOKENS)
    z_q = _gather_rows(indices, codebook)
    return (z_q, indices, loss[0])


def kernel(z, codebook):
    csq_col = jnp.sum(codebook * codebook, axis=1, keepdims=True)
    csq_row = csq_col.reshape(1, _NUM_CODES)
    zsq = jnp.sum(z * z, axis=1, keepdims=True)
    fast_ok = jnp.logical_and(jnp.min(zsq) >= 129.0,
                              jnp.max(csq_col) < _CSQ_BOUND)
    return lax.cond(
        fast_ok,
        lambda: _run(z, codebook, csq_row, zsq, True),
        lambda: _run(z, codebook, csq_row, zsq, False),
    )
